# Initial kernel scaffold; baseline (speedup 1.0000x reference)
import jax
import jax.numpy as jnp
from jax.experimental import pallas as pl


def kernel(features, edge_index, W0, b0, g0, be0, W1, b1, g1, be1, W2, b2, g2, be2, Wp, bp):
    raise NotImplementedError("write your pallas kernel here")


# speedup vs baseline: 2.8825x; 2.8825x over previous
import functools

import jax
import jax.numpy as jnp
from jax import lax
from jax.experimental import pallas as pl
from jax.experimental.pallas import tpu as pltpu
from jax.experimental.pallas import tpu_sc as plsc

NG = 1112
NPG = 9
NV = NG * NPG
NE = NV * 16
D = 256
H = 128
NC = 2
NS = 16
NP = 10240
TRASH = NV
CHUNK = 128
NCH = 40
EPAD = NC * NS * NCH * CHUNK - NE
STRIPE = NP // NS


def _mesh():
    return plsc.VectorSubcoreMesh(
        core_axis_name="c", subcore_axis_name="s", num_cores=NC, num_subcores=NS
    )


GE = NCH * CHUNK


def _deg_body(ei_hbm, cnt_hbm, idx_v, hist_v, red_v, out_v, red_sh):
    c = lax.axis_index("c")
    s = lax.axis_index("s")

    def zero(i, carry):
        hist_v[pl.ds(i * 16, 16)] = jnp.zeros((16,), jnp.float32)
        return carry

    lax.fori_loop(0, NP // 16, zero, 0)
    ones = jnp.ones((16,), jnp.float32)
    for gi in range(2):
        g = s + NS * gi
        pltpu.sync_copy(ei_hbm.at[c, g], idx_v)

        def chunk(j, carry):
            idx = idx_v[pl.ds(j * 16, 16)]
            plsc.addupdate_scatter(hist_v, [idx], ones)
            return carry

        lax.fori_loop(0, GE // 16, chunk, 0)
    pltpu.sync_copy(hist_v, red_sh.at[s])
    plsc.subcore_barrier()
    pltpu.sync_copy(red_sh.at[:, pl.ds(s * STRIPE, STRIPE)], red_v)

    def red(k, carry):
        acc = jnp.zeros((16,), jnp.float32)
        for r in range(NS):
            acc = acc + red_v[r, pl.ds(k * 16, 16)]
        out_v[pl.ds(k * 16, 16)] = acc
        return carry

    lax.fori_loop(0, STRIPE // 16, red, 0)
    pltpu.sync_copy(out_v, cnt_hbm.at[c, pl.ds(s * STRIPE, STRIPE)])


def _degrees(ei_flat):
    return pl.kernel(
        _deg_body,
        out_type=jax.ShapeDtypeStruct((NC, NP), jnp.float32),
        mesh=_mesh(),
        scratch_types=[
            pltpu.VMEM((GE,), jnp.int32),
            pltpu.VMEM((NP,), jnp.float32),
            pltpu.VMEM((NS, STRIPE), jnp.float32),
            pltpu.VMEM((STRIPE,), jnp.float32),
            pltpu.VMEM_SHARED((NS, NP), jnp.float32),
        ],
        compiler_params=pltpu.CompilerParams(needs_layout_passes=False),
    )(ei_flat)


def _spmm_body(h_hbm, ei_hbm, z_hbm, agg_hbm, sidx, didx, rows, agg_sh):
    c = lax.axis_index("c")
    s = lax.axis_index("s")
    pltpu.sync_copy(z_hbm, agg_sh.at[pl.ds(s * STRIPE, STRIPE)])
    plsc.subcore_barrier()
    for gi in range(2):
        g = s + NS * gi
        pltpu.sync_copy(ei_hbm.at[0, g], sidx)
        pltpu.sync_copy(ei_hbm.at[1, g], didx)

        def chunk(j, carry):
            pltpu.sync_copy(h_hbm.at[c].at[sidx.at[j]], rows)
            pltpu.sync_copy(rows, agg_sh.at[didx.at[j]], add=True)
            return carry

        lax.fori_loop(0, NCH, chunk, 0)
    plsc.subcore_barrier()
    pltpu.sync_copy(
        agg_sh.at[pl.ds(s * STRIPE, STRIPE)],
        agg_hbm.at[c, pl.ds(s * STRIPE, STRIPE)],
    )


def _spmm(h, ei):
    return pl.kernel(
        _spmm_body,
        out_type=jax.ShapeDtypeStruct((NC, NP, H), jnp.float32),
        mesh=_mesh(),
        scratch_types=[
            pltpu.VMEM((NCH, CHUNK), jnp.int32),
            pltpu.VMEM((NCH, CHUNK), jnp.int32),
            pltpu.VMEM((CHUNK, H), jnp.float32),
            pltpu.VMEM_SHARED((NP, H), jnp.float32),
        ],
    )(h, ei, jnp.zeros((STRIPE, H), jnp.float32))


RB = 512


def _prep_body(x_ref, co_ref, ci_ref, h_ref, sin_ref, sout_ref):
    so = lax.rsqrt(jnp.maximum(co_ref[...], 1.0))
    si = lax.rsqrt(jnp.maximum(ci_ref[...], 1.0))
    sin_ref[...] = si
    sout_ref[...] = so
    xs = x_ref[...] * so
    h_ref[0] = xs[:, :H]
    h_ref[1] = xs[:, H:]


def _prep(x, cnt_out, cnt_in):
    return pl.pallas_call(
        _prep_body,
        grid=(NP // RB,),
        in_specs=[
            pl.BlockSpec((RB, D), lambda i: (i, 0)),
            pl.BlockSpec((RB, 1), lambda i: (i, 0)),
            pl.BlockSpec((RB, 1), lambda i: (i, 0)),
        ],
        out_specs=[
            pl.BlockSpec((NC, RB, H), lambda i: (0, i, 0)),
            pl.BlockSpec((RB, 1), lambda i: (i, 0)),
            pl.BlockSpec((RB, 1), lambda i: (i, 0)),
        ],
        out_shape=[
            jax.ShapeDtypeStruct((NC, NP, H), jnp.float32),
            jax.ShapeDtypeStruct((NP, 1), jnp.float32),
            jax.ShapeDtypeStruct((NP, 1), jnp.float32),
        ],
    )(x, cnt_out, cnt_in)


def _layer_body(a_ref, sin_ref, sout_ref, w_ref, b_ref, g_ref, be_ref, h_ref):
    a = jnp.concatenate([a_ref[0], a_ref[1]], axis=1) * sin_ref[...]
    z = jnp.dot(a, w_ref[...], preferred_element_type=jnp.float32) + b_ref[...]
    mu = jnp.mean(z, axis=-1, keepdims=True)
    var = jnp.mean((z - mu) ** 2, axis=-1, keepdims=True)
    f = (z - mu) / jnp.sqrt(var + 1e-5) * g_ref[...] + be_ref[...]
    f = jnp.maximum(f, 0.0) * sout_ref[...]
    h_ref[0] = f[:, :H]
    h_ref[1] = f[:, H:]


def _layer(agg, sin, sout, w, b, g, be):
    return pl.pallas_call(
        _layer_body,
        grid=(NP // RB,),
        in_specs=[
            pl.BlockSpec((NC, RB, H), lambda i: (0, i, 0)),
            pl.BlockSpec((RB, 1), lambda i: (i, 0)),
            pl.BlockSpec((RB, 1), lambda i: (i, 0)),
            pl.BlockSpec((D, D), lambda i: (0, 0)),
            pl.BlockSpec((1, D), lambda i: (0, 0)),
            pl.BlockSpec((1, D), lambda i: (0, 0)),
            pl.BlockSpec((1, D), lambda i: (0, 0)),
        ],
        out_specs=pl.BlockSpec((NC, RB, H), lambda i: (0, i, 0)),
        out_shape=jax.ShapeDtypeStruct((NC, NP, H), jnp.float32),
    )(agg, sin, sout, w, b.reshape(1, D), g.reshape(1, D), be.reshape(1, D))


FB = 256
NGP = 1280


def _final_body(a_ref, sin_ref, w_ref, b_ref, g_ref, be_ref, wp_ref, bp_ref, y_ref):
    a = jnp.concatenate([a_ref[0], a_ref[1]], axis=1) * sin_ref[...]
    z = jnp.dot(a, w_ref[...], preferred_element_type=jnp.float32) + b_ref[...]
    mu = jnp.mean(z, axis=-1, keepdims=True)
    var = jnp.mean((z - mu) ** 2, axis=-1, keepdims=True)
    f = (z - mu) / jnp.sqrt(var + 1e-5) * g_ref[...] + be_ref[...]
    f = jnp.maximum(f, 0.0)
    y_ref[...] = (
        jnp.dot(f, wp_ref[...], preferred_element_type=jnp.float32) + bp_ref[...]
    )


def _final(sel, sin_sel, w, b, g, be, wp_pad, bp):
    return pl.pallas_call(
        _final_body,
        grid=(NGP // FB,),
        in_specs=[
            pl.BlockSpec((NC, FB, H), lambda i: (0, i, 0)),
            pl.BlockSpec((FB, 1), lambda i: (i, 0)),
            pl.BlockSpec((D, D), lambda i: (0, 0)),
            pl.BlockSpec((1, D), lambda i: (0, 0)),
            pl.BlockSpec((1, D), lambda i: (0, 0)),
            pl.BlockSpec((1, D), lambda i: (0, 0)),
            pl.BlockSpec((D, H), lambda i: (0, 0)),
            pl.BlockSpec((1, 1), lambda i: (0, 0)),
        ],
        out_specs=pl.BlockSpec((FB, H), lambda i: (i, 0)),
        out_shape=jax.ShapeDtypeStruct((NGP, H), jnp.float32),
    )(sel, sin_sel, w, b.reshape(1, D), g.reshape(1, D), be.reshape(1, D),
      wp_pad, bp.reshape(1, 1))


def kernel(features, edge_index, W0, b0, g0, be0, W1, b1, g1, be1,
           W2, b2, g2, be2, Wp, bp):
    src = edge_index[0]
    dst = edge_index[1]
    fill = jnp.full((EPAD,), TRASH, jnp.int32)
    srcp = jnp.concatenate([src, fill]).reshape(NC * NS, NCH, CHUNK)
    dstp = jnp.concatenate([dst, fill]).reshape(NC * NS, NCH, CHUNK)
    ei = jnp.stack([srcp, dstp])

    x = features.reshape(NV, D)
    x = jnp.pad(x, ((0, NP - NV), (0, 0)))

    counts = _degrees(ei.reshape(2, NC * NS, GE))
    cnt_out = counts[0][:, None]
    cnt_in = counts[1][:, None]

    h, sin, sout = _prep(x, cnt_out, cnt_in)

    Ws = [W0, W1, W2]
    bs = [b0, b1, b2]
    gs = [g0, g1, g2]
    bes = [be0, be1, be2]
    for i in range(2):
        agg = _spmm(h, ei)
        h = _layer(agg, sin, sout, Ws[i], bs[i], gs[i], bes[i])

    agg2 = _spmm(h, ei)
    sel = agg2[:, :NV, :].reshape(NC, NG, NPG, H)[:, :, 0, :]
    sel = jnp.pad(sel, ((0, 0), (0, NGP - NG), (0, 0)))
    sin_sel = sin[:NV].reshape(NG, NPG)[:, 0:1]
    sin_sel = jnp.pad(sin_sel, ((0, NGP - NG), (0, 0)))

    wp_pad = jnp.pad(Wp, ((0, 0), (0, H - 1)))
    y = _final(sel, sin_sel, W2, b2, g2, be2, wp_pad, bp)
    return y[:NG, 0:1]


# speedup vs baseline: 3.3502x; 1.1622x over previous
import functools

import jax
import jax.numpy as jnp
from jax import lax
from jax.experimental import pallas as pl
from jax.experimental.pallas import tpu as pltpu
from jax.experimental.pallas import tpu_sc as plsc

NG = 1112
NPG = 9
NV = NG * NPG
NE = NV * 16
D = 256
H = 128
NC = 2
NS = 16
NP = 10240
TRASH = NV
CHUNK = 128
NCH = 40
EPAD = NC * NS * NCH * CHUNK - NE
STRIPE = NP // NS


def _mesh():
    return plsc.VectorSubcoreMesh(
        core_axis_name="c", subcore_axis_name="s", num_cores=NC, num_subcores=NS
    )


GE = NCH * CHUNK


def _deg_body(ei_hbm, cnt_hbm, idx_v, hist_v, red_v, out_v, red_sh):
    c = lax.axis_index("c")
    s = lax.axis_index("s")

    def zero(i, carry):
        hist_v[pl.ds(i * 16, 16)] = jnp.zeros((16,), jnp.float32)
        return carry

    lax.fori_loop(0, NP // 16, zero, 0)
    ones = jnp.ones((16,), jnp.float32)
    for gi in range(2):
        g = s + NS * gi
        pltpu.sync_copy(ei_hbm.at[c, g], idx_v)

        def chunk(j, carry):
            idx = idx_v[pl.ds(j * 16, 16)]
            plsc.addupdate_scatter(hist_v, [idx], ones)
            return carry

        lax.fori_loop(0, GE // 16, chunk, 0)
    pltpu.sync_copy(hist_v, red_sh.at[s])
    plsc.subcore_barrier()
    pltpu.sync_copy(red_sh.at[:, pl.ds(s * STRIPE, STRIPE)], red_v)

    def red(k, carry):
        acc = jnp.zeros((16,), jnp.float32)
        for r in range(NS):
            acc = acc + red_v[r, pl.ds(k * 16, 16)]
        out_v[pl.ds(k * 16, 16)] = acc
        return carry

    lax.fori_loop(0, STRIPE // 16, red, 0)
    pltpu.sync_copy(out_v, cnt_hbm.at[c, pl.ds(s * STRIPE, STRIPE)])


def _degrees(ei_flat):
    return pl.kernel(
        _deg_body,
        out_type=jax.ShapeDtypeStruct((NC, NP), jnp.float32),
        mesh=_mesh(),
        scratch_types=[
            pltpu.VMEM((GE,), jnp.int32),
            pltpu.VMEM((NP,), jnp.float32),
            pltpu.VMEM((NS, STRIPE), jnp.float32),
            pltpu.VMEM((STRIPE,), jnp.float32),
            pltpu.VMEM_SHARED((NS, NP), jnp.float32),
        ],
        compiler_params=pltpu.CompilerParams(needs_layout_passes=False),
    )(ei_flat)


NT = 2 * NCH


def _spmm_body(h_hbm, ei_hbm, z_hbm, agg_hbm, sidx, didx, rows_a, rows_b,
               agg_sh, gsem_a, gsem_b):
    c = lax.axis_index("c")
    s = lax.axis_index("s")
    pltpu.sync_copy(z_hbm, agg_sh.at[pl.ds(s * STRIPE, STRIPE)])
    plsc.subcore_barrier()

    def gather(j, buf, sem):
        pltpu.async_copy(h_hbm.at[c].at[sidx.at[j]], buf, sem)

    def gwait(j, buf, sem):
        pltpu.make_async_copy(h_hbm.at[c].at[sidx.at[j]], buf, sem).wait()

    for gi in range(2):
        g = s + NS * gi
        pltpu.sync_copy(ei_hbm.at[0, g], sidx)
        pltpu.sync_copy(ei_hbm.at[1, g], didx)
        gather(0, rows_a, gsem_a)

        def body(t, carry):
            j = 2 * t
            gwait(j, rows_a, gsem_a)
            gather(j + 1, rows_b, gsem_b)
            pltpu.sync_copy(rows_a, agg_sh.at[didx.at[j]], add=True)

            @pl.when(j + 2 < NCH)
            def _():
                gather(j + 2, rows_a, gsem_a)

            gwait(j + 1, rows_b, gsem_b)
            pltpu.sync_copy(rows_b, agg_sh.at[didx.at[j + 1]], add=True)
            return carry

        lax.fori_loop(0, NCH // 2, body, 0)
    plsc.subcore_barrier()
    pltpu.sync_copy(
        agg_sh.at[pl.ds(s * STRIPE, STRIPE)],
        agg_hbm.at[c, pl.ds(s * STRIPE, STRIPE)],
    )


def _spmm(h, ei):
    return pl.kernel(
        _spmm_body,
        out_type=jax.ShapeDtypeStruct((NC, NP, H), jnp.float32),
        mesh=_mesh(),
        scratch_types=[
            pltpu.VMEM((NCH, CHUNK), jnp.int32),
            pltpu.VMEM((NCH, CHUNK), jnp.int32),
            pltpu.VMEM((CHUNK, H), jnp.float32),
            pltpu.VMEM((CHUNK, H), jnp.float32),
            pltpu.VMEM_SHARED((NP, H), jnp.float32),
            pltpu.SemaphoreType.DMA,
            pltpu.SemaphoreType.DMA,
        ],
    )(h, ei, jnp.zeros((STRIPE, H), jnp.float32))


RB = 512


def _prep_body(x_ref, co_ref, ci_ref, h_ref, sin_ref, sout_ref):
    so = lax.rsqrt(jnp.maximum(co_ref[...], 1.0))
    si = lax.rsqrt(jnp.maximum(ci_ref[...], 1.0))
    sin_ref[...] = si
    sout_ref[...] = so
    xs = x_ref[...] * so
    h_ref[0] = xs[:, :H]
    h_ref[1] = xs[:, H:]


def _prep(x, cnt_out, cnt_in):
    return pl.pallas_call(
        _prep_body,
        grid=(NP // RB,),
        in_specs=[
            pl.BlockSpec((RB, D), lambda i: (i, 0)),
            pl.BlockSpec((RB, 1), lambda i: (i, 0)),
            pl.BlockSpec((RB, 1), lambda i: (i, 0)),
        ],
        out_specs=[
            pl.BlockSpec((NC, RB, H), lambda i: (0, i, 0)),
            pl.BlockSpec((RB, 1), lambda i: (i, 0)),
            pl.BlockSpec((RB, 1), lambda i: (i, 0)),
        ],
        out_shape=[
            jax.ShapeDtypeStruct((NC, NP, H), jnp.float32),
            jax.ShapeDtypeStruct((NP, 1), jnp.float32),
            jax.ShapeDtypeStruct((NP, 1), jnp.float32),
        ],
    )(x, cnt_out, cnt_in)


def _layer_body(a_ref, sin_ref, sout_ref, w_ref, b_ref, g_ref, be_ref, h_ref):
    a = jnp.concatenate([a_ref[0], a_ref[1]], axis=1) * sin_ref[...]
    z = jnp.dot(a, w_ref[...], preferred_element_type=jnp.float32) + b_ref[...]
    mu = jnp.mean(z, axis=-1, keepdims=True)
    var = jnp.mean((z - mu) ** 2, axis=-1, keepdims=True)
    f = (z - mu) / jnp.sqrt(var + 1e-5) * g_ref[...] + be_ref[...]
    f = jnp.maximum(f, 0.0) * sout_ref[...]
    h_ref[0] = f[:, :H]
    h_ref[1] = f[:, H:]


def _layer(agg, sin, sout, w, b, g, be):
    return pl.pallas_call(
        _layer_body,
        grid=(NP // RB,),
        in_specs=[
            pl.BlockSpec((NC, RB, H), lambda i: (0, i, 0)),
            pl.BlockSpec((RB, 1), lambda i: (i, 0)),
            pl.BlockSpec((RB, 1), lambda i: (i, 0)),
            pl.BlockSpec((D, D), lambda i: (0, 0)),
            pl.BlockSpec((1, D), lambda i: (0, 0)),
            pl.BlockSpec((1, D), lambda i: (0, 0)),
            pl.BlockSpec((1, D), lambda i: (0, 0)),
        ],
        out_specs=pl.BlockSpec((NC, RB, H), lambda i: (0, i, 0)),
        out_shape=jax.ShapeDtypeStruct((NC, NP, H), jnp.float32),
    )(agg, sin, sout, w, b.reshape(1, D), g.reshape(1, D), be.reshape(1, D))


FB = 256
NGP = 1280


def _final_body(a_ref, sin_ref, w_ref, b_ref, g_ref, be_ref, wp_ref, bp_ref, y_ref):
    a = jnp.concatenate([a_ref[0], a_ref[1]], axis=1) * sin_ref[...]
    z = jnp.dot(a, w_ref[...], preferred_element_type=jnp.float32) + b_ref[...]
    mu = jnp.mean(z, axis=-1, keepdims=True)
    var = jnp.mean((z - mu) ** 2, axis=-1, keepdims=True)
    f = (z - mu) / jnp.sqrt(var + 1e-5) * g_ref[...] + be_ref[...]
    f = jnp.maximum(f, 0.0)
    y_ref[...] = (
        jnp.dot(f, wp_ref[...], preferred_element_type=jnp.float32) + bp_ref[...]
    )


def _final(sel, sin_sel, w, b, g, be, wp_pad, bp):
    return pl.pallas_call(
        _final_body,
        grid=(NGP // FB,),
        in_specs=[
            pl.BlockSpec((NC, FB, H), lambda i: (0, i, 0)),
            pl.BlockSpec((FB, 1), lambda i: (i, 0)),
            pl.BlockSpec((D, D), lambda i: (0, 0)),
            pl.BlockSpec((1, D), lambda i: (0, 0)),
            pl.BlockSpec((1, D), lambda i: (0, 0)),
            pl.BlockSpec((1, D), lambda i: (0, 0)),
            pl.BlockSpec((D, H), lambda i: (0, 0)),
            pl.BlockSpec((1, 1), lambda i: (0, 0)),
        ],
        out_specs=pl.BlockSpec((FB, H), lambda i: (i, 0)),
        out_shape=jax.ShapeDtypeStruct((NGP, H), jnp.float32),
    )(sel, sin_sel, w, b.reshape(1, D), g.reshape(1, D), be.reshape(1, D),
      wp_pad, bp.reshape(1, 1))


def kernel(features, edge_index, W0, b0, g0, be0, W1, b1, g1, be1,
           W2, b2, g2, be2, Wp, bp):
    src = edge_index[0]
    dst = edge_index[1]
    fill = jnp.full((EPAD,), TRASH, jnp.int32)
    srcp = jnp.concatenate([src, fill]).reshape(NC * NS, NCH, CHUNK)
    dstp = jnp.concatenate([dst, fill]).reshape(NC * NS, NCH, CHUNK)
    ei = jnp.stack([srcp, dstp])

    x = features.reshape(NV, D)
    x = jnp.pad(x, ((0, NP - NV), (0, 0)))

    counts = _degrees(ei.reshape(2, NC * NS, GE))
    cnt_out = counts[0][:, None]
    cnt_in = counts[1][:, None]

    h, sin, sout = _prep(x, cnt_out, cnt_in)

    Ws = [W0, W1, W2]
    bs = [b0, b1, b2]
    gs = [g0, g1, g2]
    bes = [be0, be1, be2]
    for i in range(2):
        agg = _spmm(h, ei)
        h = _layer(agg, sin, sout, Ws[i], bs[i], gs[i], bes[i])

    agg2 = _spmm(h, ei)
    sel = agg2[:, :NV, :].reshape(NC, NG, NPG, H)[:, :, 0, :]
    sel = jnp.pad(sel, ((0, 0), (0, NGP - NG), (0, 0)))
    sin_sel = sin[:NV].reshape(NG, NPG)[:, 0:1]
    sin_sel = jnp.pad(sin_sel, ((0, NGP - NG), (0, 0)))

    wp_pad = jnp.pad(Wp, ((0, 0), (0, H - 1)))
    y = _final(sel, sin_sel, W2, b2, g2, be2, wp_pad, bp)
    return y[:NG, 0:1]


# speedup vs baseline: 4.4957x; 1.3419x over previous
import functools

import jax
import jax.numpy as jnp
from jax import lax
from jax.experimental import pallas as pl
from jax.experimental.pallas import tpu as pltpu
from jax.experimental.pallas import tpu_sc as plsc

NG = 1112
NPG = 9
NV = NG * NPG
NE = NV * 16
D = 256
H = 128
NC = 2
NS = 16
NP = 10240
TRASH = NV
CHUNK = 128
NCH = 40
EPAD = NC * NS * NCH * CHUNK - NE
STRIPE = NP // NS


def _mesh():
    return plsc.VectorSubcoreMesh(
        core_axis_name="c", subcore_axis_name="s", num_cores=NC, num_subcores=NS
    )


GE = NCH * CHUNK


def _deg_body(ei_hbm, cnt_hbm, idx_v, hist_v, red_v, out_v, red_sh):
    c = lax.axis_index("c")
    s = lax.axis_index("s")

    def zero(i, carry):
        hist_v[pl.ds(i * 16, 16)] = jnp.zeros((16,), jnp.float32)
        return carry

    lax.fori_loop(0, NP // 16, zero, 0)
    ones = jnp.ones((16,), jnp.float32)
    for gi in range(2):
        g = s + NS * gi
        pltpu.sync_copy(ei_hbm.at[c, g], idx_v)

        def chunk(j, carry):
            idx = idx_v[pl.ds(j * 16, 16)]
            plsc.addupdate_scatter(hist_v, [idx], ones)
            return carry

        lax.fori_loop(0, GE // 16, chunk, 0)
    pltpu.sync_copy(hist_v, red_sh.at[s])
    plsc.subcore_barrier()
    pltpu.sync_copy(red_sh.at[:, pl.ds(s * STRIPE, STRIPE)], red_v)

    def red(k, carry):
        acc = jnp.zeros((16,), jnp.float32)
        for r in range(NS):
            acc = acc + red_v[r, pl.ds(k * 16, 16)]
        out_v[pl.ds(k * 16, 16)] = acc
        return carry

    lax.fori_loop(0, STRIPE // 16, red, 0)
    pltpu.sync_copy(out_v, cnt_hbm.at[c, pl.ds(s * STRIPE, STRIPE)])


def _degrees(ei_flat):
    return pl.kernel(
        _deg_body,
        out_type=jax.ShapeDtypeStruct((NC, NP), jnp.float32),
        mesh=_mesh(),
        scratch_types=[
            pltpu.VMEM((GE,), jnp.int32),
            pltpu.VMEM((NP,), jnp.float32),
            pltpu.VMEM((NS, STRIPE), jnp.float32),
            pltpu.VMEM((STRIPE,), jnp.float32),
            pltpu.VMEM_SHARED((NS, NP), jnp.float32),
        ],
        compiler_params=pltpu.CompilerParams(needs_layout_passes=False),
    )(ei_flat)


NT = 2 * NCH


def _spmm_body(h_hbm, ei_hbm, z_hbm, agg_hbm, sidx, didx, rows_a, rows_b,
               agg_sh, gsem_a, gsem_b, ssem_a, ssem_b):
    c = lax.axis_index("c")
    s = lax.axis_index("s")
    pltpu.sync_copy(z_hbm, agg_sh.at[pl.ds(s * STRIPE, STRIPE)])
    plsc.subcore_barrier()

    def gather(j, buf, sem):
        pltpu.async_copy(h_hbm.at[c].at[sidx.at[j]], buf, sem)

    def gwait(j, buf, sem):
        pltpu.make_async_copy(h_hbm.at[c].at[sidx.at[j]], buf, sem).wait()

    def sstart(j, buf, sem):
        pltpu.async_copy(buf, agg_sh.at[didx.at[j]], sem, add=True)

    def swait(j, buf, sem):
        pltpu.make_async_copy(buf, agg_sh.at[didx.at[j]], sem).wait()

    for gi in range(2):
        g = s + NS * gi
        pltpu.sync_copy(ei_hbm.at[0, g], sidx)
        pltpu.sync_copy(ei_hbm.at[1, g], didx)
        gather(0, rows_a, gsem_a)
        gather(1, rows_b, gsem_b)

        def body(t, carry):
            j = 2 * t
            gwait(j, rows_a, gsem_a)
            sstart(j, rows_a, ssem_a)
            gwait(j + 1, rows_b, gsem_b)
            sstart(j + 1, rows_b, ssem_b)

            @pl.when(j + 2 < NCH)
            def _():
                swait(j, rows_a, ssem_a)
                gather(j + 2, rows_a, gsem_a)
                swait(j + 1, rows_b, ssem_b)
                gather(j + 3, rows_b, gsem_b)

            return carry

        lax.fori_loop(0, NCH // 2, body, 0)
        swait(NCH - 2, rows_a, ssem_a)
        swait(NCH - 1, rows_b, ssem_b)
    plsc.subcore_barrier()
    pltpu.sync_copy(
        agg_sh.at[pl.ds(s * STRIPE, STRIPE)],
        agg_hbm.at[c, pl.ds(s * STRIPE, STRIPE)],
    )


def _spmm(h, ei):
    return pl.kernel(
        _spmm_body,
        out_type=jax.ShapeDtypeStruct((NC, NP, H), jnp.float32),
        mesh=_mesh(),
        scratch_types=[
            pltpu.VMEM((NCH, CHUNK), jnp.int32),
            pltpu.VMEM((NCH, CHUNK), jnp.int32),
            pltpu.VMEM((CHUNK, H), jnp.float32),
            pltpu.VMEM((CHUNK, H), jnp.float32),
            pltpu.VMEM_SHARED((NP, H), jnp.float32),
            pltpu.SemaphoreType.DMA,
            pltpu.SemaphoreType.DMA,
            pltpu.SemaphoreType.DMA,
            pltpu.SemaphoreType.DMA,
        ],
    )(h, ei, jnp.zeros((STRIPE, H), jnp.float32))


RB = 512


def _prep_body(x_ref, co_ref, ci_ref, h_ref, sin_ref, sout_ref):
    so = lax.rsqrt(jnp.maximum(co_ref[...], 1.0))
    si = lax.rsqrt(jnp.maximum(ci_ref[...], 1.0))
    sin_ref[...] = si
    sout_ref[...] = so
    xs = x_ref[...] * so
    h_ref[0] = xs[:, :H]
    h_ref[1] = xs[:, H:]


def _prep(x, cnt_out, cnt_in):
    return pl.pallas_call(
        _prep_body,
        grid=(NP // RB,),
        in_specs=[
            pl.BlockSpec((RB, D), lambda i: (i, 0)),
            pl.BlockSpec((RB, 1), lambda i: (i, 0)),
            pl.BlockSpec((RB, 1), lambda i: (i, 0)),
        ],
        out_specs=[
            pl.BlockSpec((NC, RB, H), lambda i: (0, i, 0)),
            pl.BlockSpec((RB, 1), lambda i: (i, 0)),
            pl.BlockSpec((RB, 1), lambda i: (i, 0)),
        ],
        out_shape=[
            jax.ShapeDtypeStruct((NC, NP, H), jnp.float32),
            jax.ShapeDtypeStruct((NP, 1), jnp.float32),
            jax.ShapeDtypeStruct((NP, 1), jnp.float32),
        ],
    )(x, cnt_out, cnt_in)


def _layer_body(a_ref, sin_ref, sout_ref, w_ref, b_ref, g_ref, be_ref, h_ref):
    a = jnp.concatenate([a_ref[0], a_ref[1]], axis=1) * sin_ref[...]
    z = jnp.dot(a, w_ref[...], preferred_element_type=jnp.float32) + b_ref[...]
    mu = jnp.mean(z, axis=-1, keepdims=True)
    var = jnp.mean((z - mu) ** 2, axis=-1, keepdims=True)
    f = (z - mu) / jnp.sqrt(var + 1e-5) * g_ref[...] + be_ref[...]
    f = jnp.maximum(f, 0.0) * sout_ref[...]
    h_ref[0] = f[:, :H]
    h_ref[1] = f[:, H:]


def _layer(agg, sin, sout, w, b, g, be):
    return pl.pallas_call(
        _layer_body,
        grid=(NP // RB,),
        in_specs=[
            pl.BlockSpec((NC, RB, H), lambda i: (0, i, 0)),
            pl.BlockSpec((RB, 1), lambda i: (i, 0)),
            pl.BlockSpec((RB, 1), lambda i: (i, 0)),
            pl.BlockSpec((D, D), lambda i: (0, 0)),
            pl.BlockSpec((1, D), lambda i: (0, 0)),
            pl.BlockSpec((1, D), lambda i: (0, 0)),
            pl.BlockSpec((1, D), lambda i: (0, 0)),
        ],
        out_specs=pl.BlockSpec((NC, RB, H), lambda i: (0, i, 0)),
        out_shape=jax.ShapeDtypeStruct((NC, NP, H), jnp.float32),
    )(agg, sin, sout, w, b.reshape(1, D), g.reshape(1, D), be.reshape(1, D))


FB = 256
NGP = 1280


def _final_body(a_ref, sin_ref, w_ref, b_ref, g_ref, be_ref, wp_ref, bp_ref, y_ref):
    a = jnp.concatenate([a_ref[0], a_ref[1]], axis=1) * sin_ref[...]
    z = jnp.dot(a, w_ref[...], preferred_element_type=jnp.float32) + b_ref[...]
    mu = jnp.mean(z, axis=-1, keepdims=True)
    var = jnp.mean((z - mu) ** 2, axis=-1, keepdims=True)
    f = (z - mu) / jnp.sqrt(var + 1e-5) * g_ref[...] + be_ref[...]
    f = jnp.maximum(f, 0.0)
    y_ref[...] = (
        jnp.dot(f, wp_ref[...], preferred_element_type=jnp.float32) + bp_ref[...]
    )


def _final(sel, sin_sel, w, b, g, be, wp_pad, bp):
    return pl.pallas_call(
        _final_body,
        grid=(NGP // FB,),
        in_specs=[
            pl.BlockSpec((NC, FB, H), lambda i: (0, i, 0)),
            pl.BlockSpec((FB, 1), lambda i: (i, 0)),
            pl.BlockSpec((D, D), lambda i: (0, 0)),
            pl.BlockSpec((1, D), lambda i: (0, 0)),
            pl.BlockSpec((1, D), lambda i: (0, 0)),
            pl.BlockSpec((1, D), lambda i: (0, 0)),
            pl.BlockSpec((D, H), lambda i: (0, 0)),
            pl.BlockSpec((1, 1), lambda i: (0, 0)),
        ],
        out_specs=pl.BlockSpec((FB, H), lambda i: (i, 0)),
        out_shape=jax.ShapeDtypeStruct((NGP, H), jnp.float32),
    )(sel, sin_sel, w, b.reshape(1, D), g.reshape(1, D), be.reshape(1, D),
      wp_pad, bp.reshape(1, 1))


def kernel(features, edge_index, W0, b0, g0, be0, W1, b1, g1, be1,
           W2, b2, g2, be2, Wp, bp):
    src = edge_index[0]
    dst = edge_index[1]
    fill = jnp.full((EPAD,), TRASH, jnp.int32)
    srcp = jnp.concatenate([src, fill]).reshape(NC * NS, NCH, CHUNK)
    dstp = jnp.concatenate([dst, fill]).reshape(NC * NS, NCH, CHUNK)
    ei = jnp.stack([srcp, dstp])

    x = features.reshape(NV, D)
    x = jnp.pad(x, ((0, NP - NV), (0, 0)))

    counts = _degrees(ei.reshape(2, NC * NS, GE))
    cnt_out = counts[0][:, None]
    cnt_in = counts[1][:, None]

    h, sin, sout = _prep(x, cnt_out, cnt_in)

    Ws = [W0, W1, W2]
    bs = [b0, b1, b2]
    gs = [g0, g1, g2]
    bes = [be0, be1, be2]
    for i in range(2):
        agg = _spmm(h, ei)
        h = _layer(agg, sin, sout, Ws[i], bs[i], gs[i], bes[i])

    agg2 = _spmm(h, ei)
    sel = agg2[:, :NV, :].reshape(NC, NG, NPG, H)[:, :, 0, :]
    sel = jnp.pad(sel, ((0, 0), (0, NGP - NG), (0, 0)))
    sin_sel = sin[:NV].reshape(NG, NPG)[:, 0:1]
    sin_sel = jnp.pad(sin_sel, ((0, NGP - NG), (0, 0)))

    wp_pad = jnp.pad(Wp, ((0, 0), (0, H - 1)))
    y = _final(sel, sin_sel, W2, b2, g2, be2, wp_pad, bp)
    return y[:NG, 0:1]
